# Initial kernel scaffold; baseline (speedup 1.0000x reference)
import jax
import jax.numpy as jnp
from jax.experimental import pallas as pl


def kernel(x, edge_index, Wl, bl, Wr, br, att, bias, lin_w, lin_b):
    raise NotImplementedError("write your pallas kernel here")


# speedup vs baseline: 50.3274x; 50.3274x over previous
import functools

import jax
import jax.numpy as jnp
from jax import lax
from jax.experimental import pallas as pl
from jax.experimental.pallas import tpu as pltpu
from jax.experimental.pallas import tpu_sc as plsc

_N = 10000
_E = 320000
_D = 128
_H = 8
_C = 16
_L = 7
_EC = 80
_NCHUNK = _E // _EC
_NW = 32
_NP = 10240
_RPT = _NP // 16

_f32 = jnp.float32


_GDN = lax.GatherDimensionNumbers(
    offset_dims=(), collapsed_slice_dims=(0,), start_index_map=(0,))


def _lane_gather(v, idx):
    return lax.gather(v, idx[:, None], _GDN, (1,),
                      mode=lax.GatherScatterMode.PROMISE_IN_BOUNDS)


def _lane_sum(v, perms):
    for p in perms:
        v = v + _lane_gather(v, p)
    return v


def _edge_alpha(i, xlb, xrb, attvs, perms):
    avs, xls = [], []
    for h in range(_H):
        xlv = xlb[i, pl.ds(h * _C, _C)]
        xrv = xrb[i, pl.ds(h * _C, _C)]
        e = xlv + xrv
        e = jnp.maximum(e, e * 0.2)
        avs.append(_lane_sum(e * attvs[h], perms))
        xls.append(xlv)
    return avs, xls


@functools.partial(
    pl.kernel,
    mesh=plsc.VectorSubcoreMesh(core_axis_name="c", subcore_axis_name="s"),
    out_type=[
        jax.ShapeDtypeStruct((2, _NP, _D), _f32),
        jax.ShapeDtypeStruct((2, _NP, 16), _f32),
    ],
    scratch_types=[
        pltpu.VMEM((_EC,), jnp.int32),
        pltpu.VMEM((_EC,), jnp.int32),
        pltpu.VMEM((_EC, _D), _f32),
        pltpu.VMEM((_EC, _D), _f32),
        pltpu.VMEM((_EC, 16), _f32),
        pltpu.VMEM((_D,), _f32),
        pltpu.VMEM((_EC,), jnp.int32),
        pltpu.VMEM_SHARED((_NP, _D), _f32),
        pltpu.VMEM_SHARED((_NP, 16), _f32),
        pltpu.SemaphoreType.DMA,
        pltpu.SemaphoreType.DMA,
    ],
)
def _sc_layer(xl_hbm, xr_hbm, src_hbm, dst_hbm, att_hbm,
              acc_out, den_out,
              srcv, dstv, xlb, xrb, denb, attv, zidx, acc, dacc, sem0, sem1):
    cid = lax.axis_index("c")
    sid = lax.axis_index("s")
    wid = sid * 2 + cid

    zv = jnp.zeros((16,), _f32)

    def zrow(i, carry):
        for t in range(_D // 16):
            xrb[i, pl.ds(t * 16, 16)] = zv
        denb[i, :] = zv
        return carry
    lax.fori_loop(0, _EC, zrow, 0)

    r0 = sid * _RPT
    lanes16 = lax.iota(jnp.int32, 16)

    def _fill_zidx(qbase):
        for t in range(_EC // 16):
            zidx[pl.ds(t * 16, 16)] = lax.broadcast(qbase + t * 16, (16,)) + lanes16

    for q in range(8):
        _fill_zidx(r0 + q * _EC)
        pltpu.sync_copy(xrb, acc.at[zidx])
        pltpu.sync_copy(denb, dacc.at[zidx])

    pltpu.sync_copy(att_hbm, attv)
    attvs = [attv[pl.ds(h * _C, _C)] for h in range(_H)]
    onehots = [jnp.where(lax.iota(jnp.int32, 16) == h, 1.0, 0.0).astype(_f32)
               for h in range(_H)]
    lanes = lax.iota(jnp.int32, 16)
    perms = [lanes ^ sh for sh in (1, 2, 4, 8)]

    pltpu.sync_copy(src_hbm.at[pl.ds(0, _EC)], srcv)
    pltpu.sync_copy(dst_hbm.at[pl.ds(0, _EC)], dstv)
    pltpu.async_copy(xl_hbm.at[srcv], xlb, sem0).wait()
    pltpu.async_copy(xr_hbm.at[dstv], xrb, sem1).wait()

    def samp_body(i, m):
        avs, _ = _edge_alpha(i, xlb, xrb, attvs, perms)
        return tuple(jnp.maximum(m[h], avs[h]) for h in range(_H))
    csh = lax.fori_loop(0, _EC, samp_body,
                        tuple(lax.broadcast(-1e30, (16,)) for _ in range(_H)))

    plsc.subcore_barrier()

    def edge_body(i, carry):
        avs, xls = _edge_alpha(i, xlb, xrb, attvs, perms)
        wfull = zv
        for h in range(_H):
            w = jnp.exp(avs[h] - csh[h])
            xlb[i, pl.ds(h * _C, _C)] = xls[h] * w
            wfull = wfull + w * onehots[h]
        denb[i, :] = wfull
        return carry

    def chunk_body(j, carry):
        k = wid + j * _NW
        base = pl.multiple_of(k * _EC, _EC)
        pltpu.sync_copy(src_hbm.at[pl.ds(base, _EC)], srcv)
        pltpu.sync_copy(dst_hbm.at[pl.ds(base, _EC)], dstv)
        pltpu.async_copy(xl_hbm.at[srcv], xlb, sem0).wait()
        pltpu.async_copy(xr_hbm.at[dstv], xrb, sem1).wait()
        lax.fori_loop(0, _EC, edge_body, 0)
        pltpu.sync_copy(xlb, acc.at[dstv], add=True)
        pltpu.sync_copy(denb, dacc.at[dstv], add=True)
        return carry

    lax.fori_loop(0, _NCHUNK // _NW, chunk_body, 0)

    plsc.subcore_barrier()

    for q in range(8):
        rr = r0 + q * _EC
        _fill_zidx(rr)
        pltpu.sync_copy(acc.at[zidx], xlb)
        pltpu.sync_copy(dacc.at[zidx], denb)
        pltpu.sync_copy(xlb, acc_out.at[cid, pl.ds(rr, _EC)])
        pltpu.sync_copy(denb, den_out.at[cid, pl.ds(rr, _EC)])


def _expand_mat():
    r = lax.broadcasted_iota(jnp.int32, (_H, _D), 0)
    c = lax.broadcasted_iota(jnp.int32, (_H, _D), 1)
    return jnp.where(c // _C == r, 1.0, 0.0).astype(_f32)


def _tc_first_body(x_ref, wl_ref, bl_ref, wr_ref, br_ref, xl_ref, xr_ref):
    xb = x_ref[...]
    xl_ref[...] = jnp.dot(xb, wl_ref[...], preferred_element_type=_f32) + bl_ref[...]
    xr_ref[...] = jnp.dot(xb, wr_ref[...], preferred_element_type=_f32) + br_ref[...]


def _tc_first(x, wl, bl2, wr, br2):
    return pl.pallas_call(
        _tc_first_body,
        grid=(80,),
        in_specs=[
            pl.BlockSpec((_EC, _D), lambda i: (i, 0)),
            pl.BlockSpec((_D, _D), lambda i: (0, 0)),
            pl.BlockSpec((1, _D), lambda i: (0, 0)),
            pl.BlockSpec((_D, _D), lambda i: (0, 0)),
            pl.BlockSpec((1, _D), lambda i: (0, 0)),
        ],
        out_specs=[pl.BlockSpec((_EC, _D), lambda i: (i, 0))] * 2,
        out_shape=[jax.ShapeDtypeStruct((_NP, _D), _f32)] * 2,
    )(x, wl, bl2, wr, br2)


def _tc_mid_body(a0_ref, a1_ref, d0_ref, d1_ref, bp_ref,
                 wl_ref, bl_ref, wr_ref, br_ref, xl_ref, xr_ref):
    den8 = d0_ref[...][:, :_H] + d1_ref[...][:, :_H] + 1e-16
    denb = jnp.dot(den8, _expand_mat(), preferred_element_type=_f32)
    h = (a0_ref[...] + a1_ref[...]) / denb + bp_ref[...]
    xl_ref[...] = jnp.dot(h, wl_ref[...], preferred_element_type=_f32) + bl_ref[...]
    xr_ref[...] = jnp.dot(h, wr_ref[...], preferred_element_type=_f32) + br_ref[...]


def _tc_mid(a0, a1, d0, d1, bp, wl, bl2, wr, br2):
    return pl.pallas_call(
        _tc_mid_body,
        grid=(80,),
        in_specs=[
            pl.BlockSpec((_EC, _D), lambda i: (i, 0)),
            pl.BlockSpec((_EC, _D), lambda i: (i, 0)),
            pl.BlockSpec((_EC, 16), lambda i: (i, 0)),
            pl.BlockSpec((_EC, 16), lambda i: (i, 0)),
            pl.BlockSpec((1, _D), lambda i: (0, 0)),
            pl.BlockSpec((_D, _D), lambda i: (0, 0)),
            pl.BlockSpec((1, _D), lambda i: (0, 0)),
            pl.BlockSpec((_D, _D), lambda i: (0, 0)),
            pl.BlockSpec((1, _D), lambda i: (0, 0)),
        ],
        out_specs=[pl.BlockSpec((_EC, _D), lambda i: (i, 0))] * 2,
        out_shape=[jax.ShapeDtypeStruct((_NP, _D), _f32)] * 2,
    )(a0, a1, d0, d1, bp, wl, bl2, wr, br2)


def _tc_final_body(a0_ref, a1_ref, d0_ref, d1_ref, bp_ref, lw_ref, lb_ref, o_ref):
    den8 = d0_ref[...][:, :_H] + d1_ref[...][:, :_H] + 1e-16
    denb = jnp.dot(den8, _expand_mat(), preferred_element_type=_f32)
    h = (a0_ref[...] + a1_ref[...]) / denb + bp_ref[...]
    o = lax.dot_general(lw_ref[...], h, (((1,), (1,)), ((), ())),
                        preferred_element_type=_f32)
    o_ref[...] = o[:, :2] + lb_ref[...]


def _tc_final(a0, a1, d0, d1, bp, lw_row, lb):
    return pl.pallas_call(
        _tc_final_body,
        out_shape=jax.ShapeDtypeStruct((1, 2), _f32),
    )(a0, a1, d0, d1, bp, lw_row, lb)


def kernel(x, edge_index, Wl, bl, Wr, br, att, bias, lin_w, lin_b):
    src = edge_index[0]
    dst = edge_index[1]
    attf = att.reshape(_L, _H * _C)

    xp = jnp.pad(x, ((0, _NP - _N), (0, 0)))
    xl, xr = _tc_first(xp, Wl[0], bl[0].reshape(1, -1), Wr[0], br[0].reshape(1, -1))
    for l in range(_L):
        acc, den = _sc_layer(xl, xr, src, dst, attf[l])
        bp = bias[l].reshape(1, -1)
        if l < _L - 1:
            xl, xr = _tc_mid(acc[0], acc[1], den[0], den[1], bp,
                             Wl[l + 1], bl[l + 1].reshape(1, -1),
                             Wr[l + 1], br[l + 1].reshape(1, -1))
        else:
            out = _tc_final(acc[0, :8], acc[1, :8], den[0, :8], den[1, :8],
                            bp, lin_w.reshape(1, -1), lin_b.reshape(1, 1))
    return out


# speedup vs baseline: 53.2877x; 1.0588x over previous
import functools

import jax
import jax.numpy as jnp
from jax import lax
from jax.experimental import pallas as pl
from jax.experimental.pallas import tpu as pltpu
from jax.experimental.pallas import tpu_sc as plsc

_N = 10000
_E = 320000
_D = 128
_H = 8
_C = 16
_L = 7
_EC = 64
_NCHUNK = _E // _EC
_NW = 32
_NP = 10240
_RPT = _NP // 16

_f32 = jnp.float32


_GDN = lax.GatherDimensionNumbers(
    offset_dims=(), collapsed_slice_dims=(0,), start_index_map=(0,))


def _lane_gather(v, idx):
    return lax.gather(v, idx[:, None], _GDN, (1,),
                      mode=lax.GatherScatterMode.PROMISE_IN_BOUNDS)


def _lane_sum(v, perms):
    for p in perms:
        v = v + _lane_gather(v, p)
    return v


def _edge_alpha(i, xlb, xrb, attvs, perms):
    avs, xls = [], []
    for h in range(_H):
        xlv = xlb[i, pl.ds(h * _C, _C)]
        xrv = xrb[i, pl.ds(h * _C, _C)]
        e = xlv + xrv
        e = jnp.maximum(e, e * 0.2)
        avs.append(_lane_sum(e * attvs[h], perms))
        xls.append(xlv)
    return avs, xls


@functools.partial(
    pl.kernel,
    mesh=plsc.VectorSubcoreMesh(core_axis_name="c", subcore_axis_name="s"),
    out_type=[
        jax.ShapeDtypeStruct((2, _NP, _D), _f32),
        jax.ShapeDtypeStruct((2, _NP, 16), _f32),
    ],
    scratch_types=[
        pltpu.VMEM((_EC,), jnp.int32),
        pltpu.VMEM((_EC,), jnp.int32),
        pltpu.VMEM((_EC,), jnp.int32),
        pltpu.VMEM((_EC,), jnp.int32),
        pltpu.VMEM((_EC, _D), _f32),
        pltpu.VMEM((_EC, _D), _f32),
        pltpu.VMEM((_EC, _D), _f32),
        pltpu.VMEM((_EC, 16), _f32),
        pltpu.VMEM((_D,), _f32),
        pltpu.VMEM((_EC,), jnp.int32),
        pltpu.VMEM_SHARED((_NP, _D), _f32),
        pltpu.VMEM_SHARED((_NP, 16), _f32),
        pltpu.SemaphoreType.DMA,
        pltpu.SemaphoreType.DMA,
        pltpu.SemaphoreType.DMA,
        pltpu.SemaphoreType.DMA,
    ],
)
def _sc_layer(xl_hbm, xr_hbm, src_hbm, dst_hbm, att_hbm,
              acc_out, den_out,
              srcv0, dstv0, srcv1, dstv1, xlb0, xrb, xlb1,
              denb, attv, zidx, acc, dacc, sem0, sem1, sem2, sem3):
    cid = lax.axis_index("c")
    sid = lax.axis_index("s")
    wid = sid * 2 + cid

    zv = jnp.zeros((16,), _f32)

    def zrow(i, carry):
        for t in range(_D // 16):
            xrb[i, pl.ds(t * 16, 16)] = zv
        denb[i, :] = zv
        return carry
    lax.fori_loop(0, _EC, zrow, 0)

    r0 = sid * _RPT
    lanes16 = lax.iota(jnp.int32, 16)

    def _fill_zidx(qbase):
        for t in range(_EC // 16):
            zidx[pl.ds(t * 16, 16)] = lax.broadcast(qbase + t * 16, (16,)) + lanes16

    for q in range(10):
        _fill_zidx(r0 + q * _EC)
        pltpu.sync_copy(xrb, acc.at[zidx])
        pltpu.sync_copy(denb, dacc.at[zidx])

    pltpu.sync_copy(att_hbm, attv)
    attvs = [attv[pl.ds(h * _C, _C)] for h in range(_H)]
    onehots = [jnp.where(lax.iota(jnp.int32, 16) == h, 1.0, 0.0).astype(_f32)
               for h in range(_H)]
    lanes = lax.iota(jnp.int32, 16)
    perms = [lanes ^ sh for sh in (1, 2, 4, 8)]

    pltpu.sync_copy(src_hbm.at[pl.ds(0, _EC)], srcv0)
    pltpu.sync_copy(dst_hbm.at[pl.ds(0, _EC)], dstv0)
    a = pltpu.async_copy(xl_hbm.at[srcv0], xlb0, sem0)
    b = pltpu.async_copy(xr_hbm.at[dstv0], xrb, sem1)
    a.wait()
    b.wait()

    def samp_body(i, m):
        avs, _ = _edge_alpha(i, xlb0, xrb, attvs, perms)
        return tuple(jnp.maximum(m[h], avs[h]) for h in range(_H))
    csh = lax.fori_loop(0, _EC, samp_body,
                        tuple(lax.broadcast(-1e30, (16,)) for _ in range(_H)))

    plsc.subcore_barrier()

    nch = jnp.where(wid < _NCHUNK - 156 * _NW, 157, 156)
    sets = ((srcv0, dstv0, xlb0, sem0),
            (srcv1, dstv1, xlb1, sem2))

    def _stage_fire(j, st):
        sv, dv, xb, sa = st
        base = pl.multiple_of((wid + j * _NW) * _EC, _EC)
        pltpu.sync_copy(src_hbm.at[pl.ds(base, _EC)], sv)
        pltpu.sync_copy(dst_hbm.at[pl.ds(base, _EC)], dv)
        pltpu.async_copy(xl_hbm.at[sv], xb, sa)

    def _consume(st):
        sv, dv, xb, sa = st
        pltpu.async_copy(xr_hbm.at[dv], xrb, sem1).wait()
        pltpu.make_async_copy(xl_hbm.at[sv], xb, sa).wait()

        def edge_body(i, carry):
            avs, xls = _edge_alpha(i, xb, xrb, attvs, perms)
            wfull = zv
            for h in range(_H):
                w = jnp.exp(avs[h] - csh[h])
                xb[i, pl.ds(h * _C, _C)] = xls[h] * w
                wfull = wfull + w * onehots[h]
            denb[i, :] = wfull
            return carry
        lax.fori_loop(0, _EC, edge_body, 0)
        pltpu.sync_copy(xb, acc.at[dv], add=True)
        pltpu.sync_copy(denb, dacc.at[dv], add=True)

    _stage_fire(0, sets[0])

    def chunk_body(j, carry):
        even = (j % 2) == 0

        @pl.when(jnp.logical_and(even, j + 1 < nch))
        def _():
            _stage_fire(j + 1, sets[1])

        @pl.when(jnp.logical_and(jnp.logical_not(even), j + 1 < nch))
        def _():
            _stage_fire(j + 1, sets[0])

        @pl.when(jnp.logical_and(even, j < nch))
        def _():
            _consume(sets[0])

        @pl.when(jnp.logical_and(jnp.logical_not(even), j < nch))
        def _():
            _consume(sets[1])
        return carry

    lax.fori_loop(0, 157, chunk_body, 0)

    plsc.subcore_barrier()

    for q in range(10):
        rr = r0 + q * _EC
        _fill_zidx(rr)
        pltpu.sync_copy(acc.at[zidx], xlb0)
        pltpu.sync_copy(dacc.at[zidx], denb)
        pltpu.sync_copy(xlb0, acc_out.at[cid, pl.ds(rr, _EC)])
        pltpu.sync_copy(denb, den_out.at[cid, pl.ds(rr, _EC)])


def _expand_mat():
    r = lax.broadcasted_iota(jnp.int32, (_H, _D), 0)
    c = lax.broadcasted_iota(jnp.int32, (_H, _D), 1)
    return jnp.where(c // _C == r, 1.0, 0.0).astype(_f32)


def _tc_first_body(x_ref, wl_ref, bl_ref, wr_ref, br_ref, xl_ref, xr_ref):
    xb = x_ref[...]
    xl_ref[...] = jnp.dot(xb, wl_ref[...], preferred_element_type=_f32) + bl_ref[...]
    xr_ref[...] = jnp.dot(xb, wr_ref[...], preferred_element_type=_f32) + br_ref[...]


def _tc_first(x, wl, bl2, wr, br2):
    return pl.pallas_call(
        _tc_first_body,
        grid=(80,),
        in_specs=[
            pl.BlockSpec((_EC, _D), lambda i: (i, 0)),
            pl.BlockSpec((_D, _D), lambda i: (0, 0)),
            pl.BlockSpec((1, _D), lambda i: (0, 0)),
            pl.BlockSpec((_D, _D), lambda i: (0, 0)),
            pl.BlockSpec((1, _D), lambda i: (0, 0)),
        ],
        out_specs=[pl.BlockSpec((_EC, _D), lambda i: (i, 0))] * 2,
        out_shape=[jax.ShapeDtypeStruct((_NP, _D), _f32)] * 2,
    )(x, wl, bl2, wr, br2)


def _tc_mid_body(a0_ref, a1_ref, d0_ref, d1_ref, bp_ref,
                 wl_ref, bl_ref, wr_ref, br_ref, xl_ref, xr_ref):
    den8 = d0_ref[...][:, :_H] + d1_ref[...][:, :_H] + 1e-16
    denb = jnp.dot(den8, _expand_mat(), preferred_element_type=_f32)
    h = (a0_ref[...] + a1_ref[...]) / denb + bp_ref[...]
    xl_ref[...] = jnp.dot(h, wl_ref[...], preferred_element_type=_f32) + bl_ref[...]
    xr_ref[...] = jnp.dot(h, wr_ref[...], preferred_element_type=_f32) + br_ref[...]


def _tc_mid(a0, a1, d0, d1, bp, wl, bl2, wr, br2):
    return pl.pallas_call(
        _tc_mid_body,
        grid=(80,),
        in_specs=[
            pl.BlockSpec((_EC, _D), lambda i: (i, 0)),
            pl.BlockSpec((_EC, _D), lambda i: (i, 0)),
            pl.BlockSpec((_EC, 16), lambda i: (i, 0)),
            pl.BlockSpec((_EC, 16), lambda i: (i, 0)),
            pl.BlockSpec((1, _D), lambda i: (0, 0)),
            pl.BlockSpec((_D, _D), lambda i: (0, 0)),
            pl.BlockSpec((1, _D), lambda i: (0, 0)),
            pl.BlockSpec((_D, _D), lambda i: (0, 0)),
            pl.BlockSpec((1, _D), lambda i: (0, 0)),
        ],
        out_specs=[pl.BlockSpec((_EC, _D), lambda i: (i, 0))] * 2,
        out_shape=[jax.ShapeDtypeStruct((_NP, _D), _f32)] * 2,
    )(a0, a1, d0, d1, bp, wl, bl2, wr, br2)


def _tc_final_body(a0_ref, a1_ref, d0_ref, d1_ref, bp_ref, lw_ref, lb_ref, o_ref):
    den8 = d0_ref[...][:, :_H] + d1_ref[...][:, :_H] + 1e-16
    denb = jnp.dot(den8, _expand_mat(), preferred_element_type=_f32)
    h = (a0_ref[...] + a1_ref[...]) / denb + bp_ref[...]
    o = lax.dot_general(lw_ref[...], h, (((1,), (1,)), ((), ())),
                        preferred_element_type=_f32)
    o_ref[...] = o[:, :2] + lb_ref[...]


def _tc_final(a0, a1, d0, d1, bp, lw_row, lb):
    return pl.pallas_call(
        _tc_final_body,
        out_shape=jax.ShapeDtypeStruct((1, 2), _f32),
    )(a0, a1, d0, d1, bp, lw_row, lb)


def kernel(x, edge_index, Wl, bl, Wr, br, att, bias, lin_w, lin_b):
    src = edge_index[0]
    dst = edge_index[1]
    attf = att.reshape(_L, _H * _C)

    xp = jnp.pad(x, ((0, _NP - _N), (0, 0)))
    xl, xr = _tc_first(xp, Wl[0], bl[0].reshape(1, -1), Wr[0], br[0].reshape(1, -1))
    for l in range(_L):
        acc, den = _sc_layer(xl, xr, src, dst, attf[l])
        bp = bias[l].reshape(1, -1)
        if l < _L - 1:
            xl, xr = _tc_mid(acc[0], acc[1], den[0], den[1], bp,
                             Wl[l + 1], bl[l + 1].reshape(1, -1),
                             Wr[l + 1], br[l + 1].reshape(1, -1))
        else:
            out = _tc_final(acc[0, :8], acc[1, :8], den[0, :8], den[1, :8],
                            bp, lin_w.reshape(1, -1), lin_b.reshape(1, 1))
    return out
